# Initial kernel scaffold; baseline (speedup 1.0000x reference)
import jax
import jax.numpy as jnp
from jax.experimental import pallas as pl


def kernel(feature_t0, feature_t1, feature_t2, feature_t3, feature_v, feature_a, umask, qmask, dia_lengths, params):
    raise NotImplementedError("write your pallas kernel here")


# speedup vs baseline: 26.9416x; 26.9416x over previous
import numpy as np
import jax
import jax.numpy as jnp
from jax import lax
from jax.experimental import pallas as pl
from jax.experimental.pallas import tpu as pltpu

SEQ = 128
NB = 32
HID = 512
DT4 = 4096
DV = 512
DA = 300
DA_PAD = 384
WIN = 10
NEG = 0.01
EPS = 1e-5
NPAIR_SLOT = 10 * SEQ

_SHIFT_IDX = np.array(
    [b * NPAIR_SLOT + (d - 1) * SEQ + i
     for b in range(NB) for d in range(1, 11) for i in range(SEQ - d)],
    dtype=np.int32)


def _lrelu(x):
    return jnp.where(x >= 0, x, NEG * x)


def _mega_kernel(valid_ref, xt_ref, fv_ref, fa_ref,
                 wt_ref, bt_ref, wv_ref, bv_ref, wa_ref, ba_ref,
                 wtv1_ref, btv1_ref, wtv2_ref, btv2_ref,
                 wta1_ref, bta1_ref, wta2_ref, bta2_ref,
                 wva1_ref, bva1_ref, wva2_ref, bva2_ref,
                 wfus_ref, bfus_ref, wes_ref, bes_ref,
                 ws1a_ref, ws1b_ref, bs1_ref, ws2_ref,
                 es_out_ref, ff_out_ref, shift_out_ref):
    f32 = jnp.float32
    vcol = valid_ref[0, 0, :][:, None]

    emo_t = _lrelu(jnp.dot(xt_ref[0], wt_ref[...],
                           preferred_element_type=f32) + bt_ref[...]) * vcol
    emo_v = _lrelu(jnp.dot(fv_ref[0], wv_ref[...],
                           preferred_element_type=f32) + bv_ref[...]) * vcol
    emo_a = _lrelu(jnp.dot(fa_ref[0], wa_ref[...],
                           preferred_element_type=f32) + ba_ref[...]) * vcol

    ii = lax.broadcasted_iota(jnp.int32, (SEQ, SEQ), 0)
    jj = lax.broadcasted_iota(jnp.int32, (SEQ, SEQ), 1)
    band = (jnp.abs(ii - jj) <= WIN).astype(f32)
    deginv = 1.0 / jnp.sum(band, axis=0)[:, None]

    def conv_pair(x1, x2, layers):
        f1, f2 = x1, x2
        a1, a2 = x1, x2
        for (w_ref, b_ref) in layers:
            w = w_ref[...]
            bb = b_ref[...]
            n1 = jnp.sqrt(jnp.sum(f1 * f1, axis=-1))[:, None]
            n2 = jnp.sqrt(jnp.sum(f2 * f2, axis=-1))[None, :]
            s = lax.dot_general(f1, f2, (((1,), (1,)), ((), ())),
                                preferred_element_type=f32)
            c = band * s / (n1 * n2 + 1e-8)
            agg1 = jnp.dot(c, f2, preferred_element_type=f32) * deginv
            agg2 = lax.dot_general(c, f1, (((0,), (0,)), ((), ())),
                                   preferred_element_type=f32) * deginv
            f1 = f1 + _lrelu(jnp.dot(agg1, w, preferred_element_type=f32) + bb)
            f2 = f2 + _lrelu(jnp.dot(agg2, w, preferred_element_type=f32) + bb)
            a1 = a1 + f1
            a2 = a2 + f2
        return a1 * (1.0 / 3.0), a2 * (1.0 / 3.0)

    tv_t, tv_v = conv_pair(emo_t, emo_v, [(wtv1_ref, btv1_ref), (wtv2_ref, btv2_ref)])
    ta_t, ta_a = conv_pair(emo_t, emo_a, [(wta1_ref, bta1_ref), (wta2_ref, bta2_ref)])
    va_v, va_a = conv_pair(emo_v, emo_a, [(wva1_ref, bva1_ref), (wva2_ref, bva2_ref)])

    stacked = jnp.concatenate([tv_t, ta_t, tv_v, va_v, ta_a, va_a], axis=0)
    fusall = _lrelu(jnp.dot(stacked, wfus_ref[...],
                            preferred_element_type=f32) + bfus_ref[...])
    ff = (fusall[0:SEQ] + fusall[SEQ:2 * SEQ] + fusall[2 * SEQ:3 * SEQ]
          + fusall[3 * SEQ:4 * SEQ] + fusall[4 * SEQ:5 * SEQ]
          + fusall[5 * SEQ:6 * SEQ]) * (1.0 / 6.0)

    es_out_ref[0] = jnp.dot(ff, wes_ref[...],
                            preferred_element_type=f32) + bes_ref[...]
    ff_out_ref[0] = ff

    p = jnp.dot(ff, ws1a_ref[...], preferred_element_type=f32)
    q = jnp.dot(ff, ws1b_ref[...], preferred_element_type=f32) + bs1_ref[...]
    hs = []
    for d in range(1, 11):
        qs = jnp.concatenate([q[d:], jnp.zeros((d, HID), f32)], axis=0)
        hs.append(_lrelu(p + qs))
    h = jnp.concatenate(hs, axis=0)
    shift_out_ref[0] = lax.dot_general(ws2_ref[...], h, (((0,), (1,)), ((), ())),
                                       preferred_element_type=f32)


def _const_spec(shape):
    return pl.BlockSpec(shape, lambda b: (0,) * len(shape))


def kernel(feature_t0, feature_t1, feature_t2, feature_t3, feature_v,
           feature_a, umask, qmask, dia_lengths, params):
    f32 = jnp.float32
    p = params

    scale = jnp.concatenate(p["bn_gamma"]) * np.float32(1.0 / np.sqrt(1.0 + EPS))
    beta = jnp.concatenate(p["bn_beta"])
    wt = p["W_t"] * scale[:, None]
    bt = (p["b_t"] + beta @ p["W_t"]).reshape(1, HID)

    xt = jnp.concatenate([feature_t0, feature_t1, feature_t2, feature_t3],
                         axis=-1).transpose(1, 0, 2)
    fv = feature_v.transpose(1, 0, 2)
    fa = jnp.pad(feature_a, ((0, 0), (0, 0), (0, DA_PAD - DA))
                 ).transpose(1, 0, 2)
    wa = jnp.pad(p["W_a"], ((0, DA_PAD - DA), (0, 0)))

    valid = (jnp.arange(SEQ)[None, :] < dia_lengths[:, None]
             ).astype(f32).reshape(NB, 1, SEQ)

    wes = jnp.zeros((HID, 128), f32)
    wes = wes.at[:, :7].set(p["W_emo"]).at[:, 7:10].set(p["W_sen"])
    bes = jnp.zeros((1, 128), f32)
    bes = bes.at[:, :7].set(p["b_emo"]).at[:, 7:10].set(p["b_sen"])
    ws1a = p["W_s1"][:HID]
    ws1b = p["W_s1"][HID:]
    ws2 = jnp.pad(p["W_s2"], ((0, 0), (0, 6)))

    conv_w = []
    for m in ("tv", "ta", "va"):
        for li in range(2):
            w, b = p[m][li]
            conv_w.extend([w, b.reshape(1, HID)])

    es, ff, shift_t = pl.pallas_call(
        _mega_kernel,
        grid=(NB,),
        in_specs=[
            pl.BlockSpec((1, 1, SEQ), lambda b: (b, 0, 0)),
            pl.BlockSpec((1, SEQ, DT4), lambda b: (b, 0, 0)),
            pl.BlockSpec((1, SEQ, DV), lambda b: (b, 0, 0)),
            pl.BlockSpec((1, SEQ, DA_PAD), lambda b: (b, 0, 0)),
            _const_spec((DT4, HID)), _const_spec((1, HID)),
            _const_spec((DV, HID)), _const_spec((1, HID)),
            _const_spec((DA_PAD, HID)), _const_spec((1, HID)),
            _const_spec((HID, HID)), _const_spec((1, HID)),
            _const_spec((HID, HID)), _const_spec((1, HID)),
            _const_spec((HID, HID)), _const_spec((1, HID)),
            _const_spec((HID, HID)), _const_spec((1, HID)),
            _const_spec((HID, HID)), _const_spec((1, HID)),
            _const_spec((HID, HID)), _const_spec((1, HID)),
            _const_spec((HID, HID)), _const_spec((1, HID)),
            _const_spec((HID, 128)), _const_spec((1, 128)),
            _const_spec((HID, HID)), _const_spec((HID, HID)),
            _const_spec((1, HID)), _const_spec((HID, 8)),
        ],
        out_specs=[
            pl.BlockSpec((1, SEQ, 128), lambda b: (b, 0, 0)),
            pl.BlockSpec((1, SEQ, HID), lambda b: (b, 0, 0)),
            pl.BlockSpec((1, 8, NPAIR_SLOT), lambda b: (b, 0, 0)),
        ],
        out_shape=[
            jax.ShapeDtypeStruct((NB, SEQ, 128), f32),
            jax.ShapeDtypeStruct((NB, SEQ, HID), f32),
            jax.ShapeDtypeStruct((NB, 8, NPAIR_SLOT), f32),
        ],
    )(valid, xt, fv, fa,
      wt, bt, p["W_v"], p["b_v"].reshape(1, HID), wa, p["b_a"].reshape(1, HID),
      *conv_w,
      p["W_fus"], p["b_fus"].reshape(1, HID), wes, bes,
      ws1a, ws1b, p["b_s1"].reshape(1, HID), ws2)

    es2 = es.reshape(NB * SEQ, 128)
    logit_emo = es2[:, :7]
    logit_sen = es2[:, 7:10]
    feat_fusion = ff.reshape(NB * SEQ, HID)
    flat = shift_t.transpose(0, 2, 1).reshape(NB * NPAIR_SLOT, 8)
    logit_shift = jnp.take(flat, jnp.asarray(_SHIFT_IDX), axis=0)[:, :2]
    return (logit_emo, logit_sen, logit_shift, feat_fusion)


# speedup vs baseline: 28.9514x; 1.0746x over previous
import numpy as np
import jax
import jax.numpy as jnp
from jax import lax
from jax.experimental import pallas as pl
from jax.experimental.pallas import tpu as pltpu

SEQ = 128
NB = 32
HID = 512
DT = 1024
DV = 512
DA = 300
DA_PAD = 384
WIN = 10
NEG = 0.01
EPS = 1e-5
NPAIR_SLOT = 10 * SEQ

_SHIFT_IDX = np.array(
    [b * NPAIR_SLOT + (d - 1) * SEQ + i
     for b in range(NB) for d in range(1, 11) for i in range(SEQ - d)],
    dtype=np.int32)


def _lrelu(x):
    return jnp.where(x >= 0, x, NEG * x)


def _mega_kernel(valid_ref, ft0_ref, ft1_ref, ft2_ref, ft3_ref, fv_ref, fa_ref,
                 wt_ref, bt_ref, wv_ref, bv_ref, wa_ref, ba_ref,
                 wtv1_ref, btv1_ref, wtv2_ref, btv2_ref,
                 wta1_ref, bta1_ref, wta2_ref, bta2_ref,
                 wva1_ref, bva1_ref, wva2_ref, bva2_ref,
                 wfus_ref, bfus_ref, wes_ref, bes_ref,
                 ws1a_ref, ws1b_ref, bs1_ref, ws2_ref,
                 es_out_ref, ff_out_ref, shift_out_ref):
    f32 = jnp.float32
    vcol = valid_ref[0, 0, :][:, None]

    wt = wt_ref[...]
    pre_t = (jnp.dot(ft0_ref[...], wt[0:DT], preferred_element_type=f32)
             + jnp.dot(ft1_ref[...], wt[DT:2 * DT], preferred_element_type=f32)
             + jnp.dot(ft2_ref[...], wt[2 * DT:3 * DT], preferred_element_type=f32)
             + jnp.dot(ft3_ref[...], wt[3 * DT:4 * DT], preferred_element_type=f32))
    emo_t = _lrelu(pre_t + bt_ref[...]) * vcol
    emo_v = _lrelu(jnp.dot(fv_ref[...], wv_ref[...],
                           preferred_element_type=f32) + bv_ref[...]) * vcol
    emo_a = _lrelu(jnp.dot(fa_ref[...], wa_ref[...],
                           preferred_element_type=f32) + ba_ref[...]) * vcol

    ii = lax.broadcasted_iota(jnp.int32, (SEQ, SEQ), 0)
    jj = lax.broadcasted_iota(jnp.int32, (SEQ, SEQ), 1)
    band = (jnp.abs(ii - jj) <= WIN).astype(f32)
    deginv = 1.0 / jnp.sum(band, axis=0)[:, None]
    ones_col = jnp.ones((HID, 8), f32)
    ones_row = jnp.ones((8, HID), f32)

    def conv_pair(x1, x2, layers):
        f1, f2 = x1, x2
        a1, a2 = x1, x2
        for (w_ref, b_ref) in layers:
            w = w_ref[...]
            bb = b_ref[...]
            sq1 = jnp.dot(f1 * f1, ones_col, preferred_element_type=f32)[:, 0:1]
            sq2 = lax.dot_general(ones_row, f2 * f2, (((1,), (1,)), ((), ())),
                                  preferred_element_type=f32)[0:1, :]
            rn1 = lax.rsqrt(sq1 + 1e-16)
            rn2 = lax.rsqrt(sq2 + 1e-16)
            s = lax.dot_general(f1, f2, (((1,), (1,)), ((), ())),
                                preferred_element_type=f32)
            c = (band * s) * (rn1 * rn2)
            agg1 = jnp.dot(c * deginv, f2, preferred_element_type=f32)
            agg2 = lax.dot_general(c * deginv[:, 0][None, :], f1,
                                   (((0,), (0,)), ((), ())),
                                   preferred_element_type=f32)
            f1 = f1 + _lrelu(jnp.dot(agg1, w, preferred_element_type=f32) + bb)
            f2 = f2 + _lrelu(jnp.dot(agg2, w, preferred_element_type=f32) + bb)
            a1 = a1 + f1
            a2 = a2 + f2
        return a1 * (1.0 / 3.0), a2 * (1.0 / 3.0)

    tv_t, tv_v = conv_pair(emo_t, emo_v, [(wtv1_ref, btv1_ref), (wtv2_ref, btv2_ref)])
    ta_t, ta_a = conv_pair(emo_t, emo_a, [(wta1_ref, bta1_ref), (wta2_ref, bta2_ref)])
    va_v, va_a = conv_pair(emo_v, emo_a, [(wva1_ref, bva1_ref), (wva2_ref, bva2_ref)])

    stacked = jnp.concatenate([tv_t, ta_t, tv_v, va_v, ta_a, va_a], axis=0)
    fusall = _lrelu(jnp.dot(stacked, wfus_ref[...],
                            preferred_element_type=f32) + bfus_ref[...])
    ff = (fusall[0:SEQ] + fusall[SEQ:2 * SEQ] + fusall[2 * SEQ:3 * SEQ]
          + fusall[3 * SEQ:4 * SEQ] + fusall[4 * SEQ:5 * SEQ]
          + fusall[5 * SEQ:6 * SEQ]) * (1.0 / 6.0)

    es_out_ref[0] = jnp.dot(ff, wes_ref[...],
                            preferred_element_type=f32) + bes_ref[...]
    ff_out_ref[0] = ff

    p = jnp.dot(ff, ws1a_ref[...], preferred_element_type=f32)
    q = jnp.dot(ff, ws1b_ref[...], preferred_element_type=f32) + bs1_ref[...]
    qpad = jnp.concatenate([q, jnp.zeros((16, HID), f32)], axis=0)
    hs = [_lrelu(p + qpad[d:d + SEQ]) for d in range(1, 11)]
    h = jnp.concatenate(hs, axis=0)
    shift_out_ref[0] = jnp.dot(h, ws2_ref[...],
                               preferred_element_type=f32)


def _const_spec(shape):
    return pl.BlockSpec(shape, lambda b: (0,) * len(shape))


def kernel(feature_t0, feature_t1, feature_t2, feature_t3, feature_v,
           feature_a, umask, qmask, dia_lengths, params):
    f32 = jnp.float32
    p = params

    scale = jnp.concatenate(p["bn_gamma"]) * np.float32(1.0 / np.sqrt(1.0 + EPS))
    beta = jnp.concatenate(p["bn_beta"])
    wt = p["W_t"] * scale[:, None]
    bt = (p["b_t"] + beta @ p["W_t"]).reshape(1, HID)

    ft0 = feature_t0.reshape(SEQ, NB * DT)
    ft1 = feature_t1.reshape(SEQ, NB * DT)
    ft2 = feature_t2.reshape(SEQ, NB * DT)
    ft3 = feature_t3.reshape(SEQ, NB * DT)
    fv = feature_v.reshape(SEQ, NB * DV)
    fa = jnp.pad(feature_a, ((0, 0), (0, 0), (0, DA_PAD - DA))
                 ).reshape(SEQ, NB * DA_PAD)
    wa = jnp.pad(p["W_a"], ((0, DA_PAD - DA), (0, 0)))

    valid = (jnp.arange(SEQ)[None, :] < dia_lengths[:, None]
             ).astype(f32).reshape(NB, 1, SEQ)

    wes = jnp.zeros((HID, 128), f32)
    wes = wes.at[:, :7].set(p["W_emo"]).at[:, 7:10].set(p["W_sen"])
    bes = jnp.zeros((1, 128), f32)
    bes = bes.at[:, :7].set(p["b_emo"]).at[:, 7:10].set(p["b_sen"])
    ws1a = p["W_s1"][:HID]
    ws1b = p["W_s1"][HID:]
    ws2 = jnp.pad(p["W_s2"], ((0, 0), (0, 6)))

    conv_w = []
    for m in ("tv", "ta", "va"):
        for li in range(2):
            w, b = p[m][li]
            conv_w.extend([w, b.reshape(1, HID)])

    es, ff, shift_flat = pl.pallas_call(
        _mega_kernel,
        grid=(NB,),
        in_specs=[
            pl.BlockSpec((1, 1, SEQ), lambda b: (b, 0, 0)),
            pl.BlockSpec((SEQ, DT), lambda b: (0, b)),
            pl.BlockSpec((SEQ, DT), lambda b: (0, b)),
            pl.BlockSpec((SEQ, DT), lambda b: (0, b)),
            pl.BlockSpec((SEQ, DT), lambda b: (0, b)),
            pl.BlockSpec((SEQ, DV), lambda b: (0, b)),
            pl.BlockSpec((SEQ, DA_PAD), lambda b: (0, b)),
            _const_spec((4 * DT, HID)), _const_spec((1, HID)),
            _const_spec((DV, HID)), _const_spec((1, HID)),
            _const_spec((DA_PAD, HID)), _const_spec((1, HID)),
            _const_spec((HID, HID)), _const_spec((1, HID)),
            _const_spec((HID, HID)), _const_spec((1, HID)),
            _const_spec((HID, HID)), _const_spec((1, HID)),
            _const_spec((HID, HID)), _const_spec((1, HID)),
            _const_spec((HID, HID)), _const_spec((1, HID)),
            _const_spec((HID, HID)), _const_spec((1, HID)),
            _const_spec((HID, HID)), _const_spec((1, HID)),
            _const_spec((HID, 128)), _const_spec((1, 128)),
            _const_spec((HID, HID)), _const_spec((HID, HID)),
            _const_spec((1, HID)), _const_spec((HID, 8)),
        ],
        out_specs=[
            pl.BlockSpec((1, SEQ, 128), lambda b: (b, 0, 0)),
            pl.BlockSpec((1, SEQ, HID), lambda b: (b, 0, 0)),
            pl.BlockSpec((1, NPAIR_SLOT, 8), lambda b: (b, 0, 0)),
        ],
        out_shape=[
            jax.ShapeDtypeStruct((NB, SEQ, 128), f32),
            jax.ShapeDtypeStruct((NB, SEQ, HID), f32),
            jax.ShapeDtypeStruct((NB, NPAIR_SLOT, 8), f32),
        ],
    )(valid, ft0, ft1, ft2, ft3, fv, fa,
      wt, bt, p["W_v"], p["b_v"].reshape(1, HID), wa, p["b_a"].reshape(1, HID),
      *conv_w,
      p["W_fus"], p["b_fus"].reshape(1, HID), wes, bes,
      ws1a, ws1b, p["b_s1"].reshape(1, HID), ws2)

    es2 = es.reshape(NB * SEQ, 128)
    logit_emo = es2[:, :7]
    logit_sen = es2[:, 7:10]
    feat_fusion = ff.reshape(NB * SEQ, HID)
    flat = shift_flat.reshape(NB * NPAIR_SLOT, 8)
    logit_shift = jnp.take(flat, jnp.asarray(_SHIFT_IDX), axis=0)[:, :2]
    return (logit_emo, logit_sen, logit_shift, feat_fusion)


# speedup vs baseline: 32.7898x; 1.1326x over previous
import numpy as np
import jax
import jax.numpy as jnp
from jax import lax
from jax.experimental import pallas as pl
from jax.experimental.pallas import tpu as pltpu

SEQ = 128
NB = 32
N = SEQ * NB
HID = 512
DT = 1024
DV = 512
DA = 300
WIN = 10
NEG = 0.01
EPS = 1e-5
ROWS = 512
NPAIR_SLOT = 10 * SEQ

_SHIFT_IDX = np.array(
    [b * NPAIR_SLOT + (d - 1) * SEQ + i
     for b in range(NB) for d in range(1, 11) for i in range(SEQ - d)],
    dtype=np.int32)


def _lrelu(x):
    return jnp.where(x >= 0, x, NEG * x)


def _proj_kernel(valid_ref, ft0_ref, ft1_ref, ft2_ref, ft3_ref, fv_ref, fa_ref,
                 wt_ref, bt_ref, wv_ref, bv_ref, wa_ref, ba_ref,
                 et_ref, ev_ref, ea_ref):
    f32 = jnp.float32
    v = valid_ref[...]
    wt = wt_ref[...]
    pre = (jnp.dot(ft0_ref[...], wt[0:DT], preferred_element_type=f32)
           + jnp.dot(ft1_ref[...], wt[DT:2 * DT], preferred_element_type=f32)
           + jnp.dot(ft2_ref[...], wt[2 * DT:3 * DT], preferred_element_type=f32)
           + jnp.dot(ft3_ref[...], wt[3 * DT:4 * DT], preferred_element_type=f32))
    et_ref[...] = _lrelu(pre + bt_ref[...]) * v
    ev_ref[...] = _lrelu(jnp.dot(fv_ref[...], wv_ref[...],
                                 preferred_element_type=f32) + bv_ref[...]) * v
    ea_ref[...] = _lrelu(jnp.dot(fa_ref[...], wa_ref[...],
                                 preferred_element_type=f32) + ba_ref[...]) * v


def _graph_kernel(et_ref, ev_ref, ea_ref,
                  wtv1_ref, btv1_ref, wtv2_ref, btv2_ref,
                  wta1_ref, bta1_ref, wta2_ref, bta2_ref,
                  wva1_ref, bva1_ref, wva2_ref, bva2_ref,
                  wfus_ref, bfus_ref, wes_ref, bes_ref,
                  ws1a_ref, ws1b_ref, bs1_ref, ws2_ref,
                  es_out_ref, ff_out_ref, shift_out_ref):
    f32 = jnp.float32
    emo_t = et_ref[0]
    emo_v = ev_ref[0]
    emo_a = ea_ref[0]

    ii = lax.broadcasted_iota(jnp.int32, (SEQ, SEQ), 0)
    jj = lax.broadcasted_iota(jnp.int32, (SEQ, SEQ), 1)
    band = (jnp.abs(ii - jj) <= WIN).astype(f32)
    deginv = 1.0 / jnp.sum(band, axis=0)[:, None]
    ones_col = jnp.ones((HID, 8), f32)
    ones_row = jnp.ones((8, HID), f32)

    def conv_pair(x1, x2, layers):
        f1, f2 = x1, x2
        a1, a2 = x1, x2
        for (w_ref, b_ref) in layers:
            w = w_ref[...]
            bb = b_ref[...]
            sq1 = jnp.dot(f1 * f1, ones_col, preferred_element_type=f32)[:, 0:1]
            sq2 = lax.dot_general(ones_row, f2 * f2, (((1,), (1,)), ((), ())),
                                  preferred_element_type=f32)[0:1, :]
            rn1 = lax.rsqrt(sq1 + 1e-16)
            rn2 = lax.rsqrt(sq2 + 1e-16)
            s = lax.dot_general(f1, f2, (((1,), (1,)), ((), ())),
                                preferred_element_type=f32)
            c = (band * s) * (rn1 * rn2)
            agg1 = jnp.dot(c * deginv, f2, preferred_element_type=f32)
            agg2 = lax.dot_general(c * deginv[:, 0][None, :], f1,
                                   (((0,), (0,)), ((), ())),
                                   preferred_element_type=f32)
            f1 = f1 + _lrelu(jnp.dot(agg1, w, preferred_element_type=f32) + bb)
            f2 = f2 + _lrelu(jnp.dot(agg2, w, preferred_element_type=f32) + bb)
            a1 = a1 + f1
            a2 = a2 + f2
        return a1 * (1.0 / 3.0), a2 * (1.0 / 3.0)

    tv_t, tv_v = conv_pair(emo_t, emo_v, [(wtv1_ref, btv1_ref), (wtv2_ref, btv2_ref)])
    ta_t, ta_a = conv_pair(emo_t, emo_a, [(wta1_ref, bta1_ref), (wta2_ref, bta2_ref)])
    va_v, va_a = conv_pair(emo_v, emo_a, [(wva1_ref, bva1_ref), (wva2_ref, bva2_ref)])

    stacked = jnp.concatenate([tv_t, ta_t, tv_v, va_v, ta_a, va_a], axis=0)
    fusall = _lrelu(jnp.dot(stacked, wfus_ref[...],
                            preferred_element_type=f32) + bfus_ref[...])
    ff = (fusall[0:SEQ] + fusall[SEQ:2 * SEQ] + fusall[2 * SEQ:3 * SEQ]
          + fusall[3 * SEQ:4 * SEQ] + fusall[4 * SEQ:5 * SEQ]
          + fusall[5 * SEQ:6 * SEQ]) * (1.0 / 6.0)

    es_out_ref[0] = jnp.dot(ff, wes_ref[...],
                            preferred_element_type=f32) + bes_ref[...]
    ff_out_ref[0] = ff

    p = jnp.dot(ff, ws1a_ref[...], preferred_element_type=f32)
    q = jnp.dot(ff, ws1b_ref[...], preferred_element_type=f32) + bs1_ref[...]
    qpad = jnp.concatenate([q, jnp.zeros((16, HID), f32)], axis=0)
    hs = [_lrelu(p + qpad[d:d + SEQ]) for d in range(1, 11)]
    h = jnp.concatenate(hs, axis=0)
    shift_out_ref[0] = jnp.dot(h, ws2_ref[...],
                               preferred_element_type=f32)


def _const_spec(shape):
    return pl.BlockSpec(shape, lambda b: (0,) * len(shape))


def kernel(feature_t0, feature_t1, feature_t2, feature_t3, feature_v,
           feature_a, umask, qmask, dia_lengths, params):
    f32 = jnp.float32
    p = params

    scale = jnp.concatenate(p["bn_gamma"]) * np.float32(1.0 / np.sqrt(1.0 + EPS))
    beta = jnp.concatenate(p["bn_beta"])
    wt = p["W_t"] * scale[:, None]
    bt = (p["b_t"] + beta @ p["W_t"]).reshape(1, HID)

    ft0 = feature_t0.reshape(N, DT)
    ft1 = feature_t1.reshape(N, DT)
    ft2 = feature_t2.reshape(N, DT)
    ft3 = feature_t3.reshape(N, DT)
    fv = feature_v.reshape(N, DV)
    fa = feature_a.reshape(N, DA)

    valid = (jnp.arange(SEQ)[:, None] < dia_lengths[None, :]
             ).astype(f32).reshape(N, 1)

    et, ev, ea = pl.pallas_call(
        _proj_kernel,
        grid=(N // ROWS,),
        in_specs=[
            pl.BlockSpec((ROWS, 1), lambda i: (i, 0)),
            pl.BlockSpec((ROWS, DT), lambda i: (i, 0)),
            pl.BlockSpec((ROWS, DT), lambda i: (i, 0)),
            pl.BlockSpec((ROWS, DT), lambda i: (i, 0)),
            pl.BlockSpec((ROWS, DT), lambda i: (i, 0)),
            pl.BlockSpec((ROWS, DV), lambda i: (i, 0)),
            pl.BlockSpec((ROWS, DA), lambda i: (i, 0)),
            _const_spec((4 * DT, HID)), _const_spec((1, HID)),
            _const_spec((DV, HID)), _const_spec((1, HID)),
            _const_spec((DA, HID)), _const_spec((1, HID)),
        ],
        out_specs=[
            pl.BlockSpec((ROWS, HID), lambda i: (i, 0)),
            pl.BlockSpec((ROWS, HID), lambda i: (i, 0)),
            pl.BlockSpec((ROWS, HID), lambda i: (i, 0)),
        ],
        out_shape=[jax.ShapeDtypeStruct((N, HID), f32)] * 3,
    )(valid, ft0, ft1, ft2, ft3, fv, fa,
      wt, bt, p["W_v"], p["b_v"].reshape(1, HID), p["W_a"],
      p["b_a"].reshape(1, HID))

    etd = et.reshape(SEQ, NB, HID).transpose(1, 0, 2)
    evd = ev.reshape(SEQ, NB, HID).transpose(1, 0, 2)
    ead = ea.reshape(SEQ, NB, HID).transpose(1, 0, 2)

    wes = jnp.zeros((HID, 128), f32)
    wes = wes.at[:, :7].set(p["W_emo"]).at[:, 7:10].set(p["W_sen"])
    bes = jnp.zeros((1, 128), f32)
    bes = bes.at[:, :7].set(p["b_emo"]).at[:, 7:10].set(p["b_sen"])
    ws1a = p["W_s1"][:HID]
    ws1b = p["W_s1"][HID:]
    ws2 = jnp.pad(p["W_s2"], ((0, 0), (0, 6)))

    conv_w = []
    for m in ("tv", "ta", "va"):
        for li in range(2):
            w, b = p[m][li]
            conv_w.extend([w, b.reshape(1, HID)])

    es, ff, shift_flat = pl.pallas_call(
        _graph_kernel,
        grid=(NB,),
        in_specs=[
            pl.BlockSpec((1, SEQ, HID), lambda b: (b, 0, 0)),
            pl.BlockSpec((1, SEQ, HID), lambda b: (b, 0, 0)),
            pl.BlockSpec((1, SEQ, HID), lambda b: (b, 0, 0)),
            _const_spec((HID, HID)), _const_spec((1, HID)),
            _const_spec((HID, HID)), _const_spec((1, HID)),
            _const_spec((HID, HID)), _const_spec((1, HID)),
            _const_spec((HID, HID)), _const_spec((1, HID)),
            _const_spec((HID, HID)), _const_spec((1, HID)),
            _const_spec((HID, HID)), _const_spec((1, HID)),
            _const_spec((HID, HID)), _const_spec((1, HID)),
            _const_spec((HID, 128)), _const_spec((1, 128)),
            _const_spec((HID, HID)), _const_spec((HID, HID)),
            _const_spec((1, HID)), _const_spec((HID, 8)),
        ],
        out_specs=[
            pl.BlockSpec((1, SEQ, 128), lambda b: (b, 0, 0)),
            pl.BlockSpec((1, SEQ, HID), lambda b: (b, 0, 0)),
            pl.BlockSpec((1, NPAIR_SLOT, 8), lambda b: (b, 0, 0)),
        ],
        out_shape=[
            jax.ShapeDtypeStruct((NB, SEQ, 128), f32),
            jax.ShapeDtypeStruct((NB, SEQ, HID), f32),
            jax.ShapeDtypeStruct((NB, NPAIR_SLOT, 8), f32),
        ],
    )(etd, evd, ead,
      *conv_w,
      p["W_fus"], p["b_fus"].reshape(1, HID), wes, bes,
      ws1a, ws1b, p["b_s1"].reshape(1, HID), ws2)

    es2 = es.reshape(NB * SEQ, 128)
    logit_emo = es2[:, :7]
    logit_sen = es2[:, 7:10]
    feat_fusion = ff.reshape(NB * SEQ, HID)
    flat = shift_flat.reshape(NB * NPAIR_SLOT, 8)
    logit_shift = jnp.take(flat, jnp.asarray(_SHIFT_IDX), axis=0)[:, :2]
    return (logit_emo, logit_sen, logit_shift, feat_fusion)
